# baseline (speedup 1.0000x reference)
import functools

import jax
import jax.numpy as jnp
from jax import lax
from jax.experimental import pallas as pl
from jax.experimental.pallas import tpu as pltpu
from jax.experimental.pallas import tpu_sc as plsc

B = 16384
D = 64
NC, NS, L = 2, 16, 16
NW = NC * NS
BPW = B // NW
CH = 128
NCH = BPW // CH


def _sc_body(users_h, pos_h, neg_h, ut_h, it_h, s_out, regp_out,
             uidx, pidx, nidx, urows, prows, nrows, sbuf, regbuf, sem):
    wid = lax.axis_index("s") * NC + lax.axis_index("c")
    base = wid * BPW

    pltpu.sync_copy(users_h.at[pl.ds(base, BPW)], uidx)
    pltpu.sync_copy(pos_h.at[pl.ds(base, BPW)], pidx)
    pltpu.sync_copy(neg_h.at[pl.ds(base, BPW)], nidx)

    descs = []
    for tab, idxr, dst in ((ut_h, uidx, urows),
                           (it_h, pidx, prows),
                           (it_h, nidx, nrows)):
        for j in range(NCH):
            sl = pl.ds(j * CH, CH)
            descs.append(pltpu.async_copy(tab.at[idxr.at[sl]], dst.at[sl], sem))
    for dsc in descs:
        dsc.wait()

    def reg_body(r, acc):
        for d in range(D // L):
            sl = pl.ds(d * L, L)
            u = urows[r, sl]
            p = prows[r, sl]
            n = nrows[r, sl]
            acc = acc + (u * u + (p * p + n * n))
            nrows[r, sl] = n - p
        return acc

    acc = lax.fori_loop(0, BPW, reg_body, jnp.zeros((L,), jnp.float32))
    regbuf[...] = acc

    lanes = lax.iota(jnp.int32, L)

    def score_body(b, _):
        rows = b * L + lanes
        s = jnp.zeros((L,), jnp.float32)
        for d in range(D):
            col = jnp.full((L,), d, jnp.int32)
            s = s + plsc.load_gather(urows, [rows, col]) * \
                plsc.load_gather(nrows, [rows, col])
        sbuf[pl.ds(b * L, L)] = s
        return 0

    lax.fori_loop(0, BPW // L, score_body, 0)

    pltpu.sync_copy(sbuf, s_out.at[pl.ds(base, BPW)])
    pltpu.sync_copy(regbuf, regp_out.at[wid])


_sc_call = functools.partial(
    pl.kernel,
    out_type=(jax.ShapeDtypeStruct((B,), jnp.float32),
              jax.ShapeDtypeStruct((NW, L), jnp.float32)),
    mesh=plsc.VectorSubcoreMesh(core_axis_name="c", subcore_axis_name="s",
                                num_cores=NC, num_subcores=NS),
    compiler_params=pltpu.CompilerParams(needs_layout_passes=False,
                                         use_tc_tiling_on_sc=False),
    scratch_types=(
        pltpu.VMEM((BPW,), jnp.int32),
        pltpu.VMEM((BPW,), jnp.int32),
        pltpu.VMEM((BPW,), jnp.int32),
        pltpu.VMEM((BPW, D), jnp.float32),
        pltpu.VMEM((BPW, D), jnp.float32),
        pltpu.VMEM((BPW, D), jnp.float32),
        pltpu.VMEM((BPW,), jnp.float32),
        pltpu.VMEM((L,), jnp.float32),
        pltpu.SemaphoreType.DMA,
    ),
)(_sc_body)


def _tc_body(s_ref, regp_ref, loss_ref, reg_ref):
    s = s_ref[...]
    loss_ref[0, 0] = jnp.sum(jax.nn.softplus(s)) * (1.0 / B)
    reg_ref[0, 0] = jnp.sum(regp_ref[...]) * (0.5 / B)


def kernel(users, pos, neg, user_table, item_table):
    s, regp = _sc_call(users, pos, neg, user_table, item_table)
    loss, reg = pl.pallas_call(
        _tc_body,
        out_shape=(jax.ShapeDtypeStruct((1, 1), jnp.float32),
                   jax.ShapeDtypeStruct((1, 1), jnp.float32)),
        out_specs=(pl.BlockSpec(memory_space=pltpu.SMEM),
                   pl.BlockSpec(memory_space=pltpu.SMEM)),
    )(s.reshape(B // 128, 128), regp)
    return (loss[0, 0], reg[0, 0])


# speedup vs baseline: 1.5155x; 1.5155x over previous
import functools

import jax
import jax.numpy as jnp
from jax import lax
from jax.experimental import pallas as pl
from jax.experimental.pallas import tpu as pltpu
from jax.experimental.pallas import tpu_sc as plsc

B = 16384
D = 64
NC, NS, L = 2, 16, 16
NW = NC * NS
BPW = B // NW
CH = 128
NCH = BPW // CH


def _sc_body(users_h, pos_h, neg_h, ut_h, it_h, s_out, regp_out,
             sidx_u, sidx_p, sidx_n, vidx, ubuf, pbuf, nbuf,
             sbuf, regbuf, semu, semp, semn):
    wid = lax.axis_index("s") * NC + lax.axis_index("c")
    base = wid * BPW

    for src_h, dst_s in ((users_h, sidx_u), (pos_h, sidx_p), (neg_h, sidx_n)):
        pltpu.sync_copy(src_h.at[pl.ds(base, BPW)], vidx)

        def spill(g, _, dst_s=dst_s):
            v = vidx[pl.ds(g * L, L)]
            for lane in range(L):
                dst_s[g * L + lane] = v[lane]
            return 0

        lax.fori_loop(0, BPW // L, spill, 0)

    lanes = lax.iota(jnp.int32, L)

    def chunk_body(c, acc_reg):
        cb = c * CH

        def fetch(r, _):
            pltpu.async_copy(ut_h.at[sidx_u[cb + r]], ubuf.at[r], semu)
            pltpu.async_copy(it_h.at[sidx_p[cb + r]], pbuf.at[r], semp)
            pltpu.async_copy(it_h.at[sidx_n[cb + r]], nbuf.at[r], semn)
            return 0

        lax.fori_loop(0, CH, fetch, 0)

        def drain(r, _):
            pltpu.make_async_copy(ut_h.at[0], ubuf.at[r], semu).wait()
            pltpu.make_async_copy(it_h.at[0], pbuf.at[r], semp).wait()
            pltpu.make_async_copy(it_h.at[0], nbuf.at[r], semn).wait()
            return 0

        lax.fori_loop(0, CH, drain, 0)

        def group(g, acc):
            rows = g * L + lanes
            acc_s = jnp.zeros((L,), jnp.float32)
            for d in range(D):
                dcol = jnp.full((L,), d, jnp.int32)
                uv = plsc.load_gather(ubuf, [rows, dcol])
                pv = plsc.load_gather(pbuf, [rows, dcol])
                nv = plsc.load_gather(nbuf, [rows, dcol])
                acc_s = acc_s + uv * (nv - pv)
                acc = acc + (uv * uv + (pv * pv + nv * nv))
            sbuf[pl.ds(cb + g * L, L)] = acc_s
            return acc

        return lax.fori_loop(0, CH // L, group, acc_reg)

    acc_reg = lax.fori_loop(0, NCH, chunk_body, jnp.zeros((L,), jnp.float32))
    regbuf[...] = acc_reg

    pltpu.sync_copy(sbuf, s_out.at[pl.ds(base, BPW)])
    pltpu.sync_copy(regbuf, regp_out.at[wid])


_sc_call = functools.partial(
    pl.kernel,
    out_type=(jax.ShapeDtypeStruct((B,), jnp.float32),
              jax.ShapeDtypeStruct((NW, L), jnp.float32)),
    mesh=plsc.VectorSubcoreMesh(core_axis_name="c", subcore_axis_name="s",
                                num_cores=NC, num_subcores=NS),
    compiler_params=pltpu.CompilerParams(needs_layout_passes=False),
    scratch_types=(
        pltpu.SMEM((BPW,), jnp.int32),
        pltpu.SMEM((BPW,), jnp.int32),
        pltpu.SMEM((BPW,), jnp.int32),
        pltpu.VMEM((BPW,), jnp.int32),
        pltpu.VMEM((CH, D), jnp.float32),
        pltpu.VMEM((CH, D), jnp.float32),
        pltpu.VMEM((CH, D), jnp.float32),
        pltpu.VMEM((BPW,), jnp.float32),
        pltpu.VMEM((L,), jnp.float32),
        pltpu.SemaphoreType.DMA,
        pltpu.SemaphoreType.DMA,
        pltpu.SemaphoreType.DMA,
    ),
)(_sc_body)


def _tc_body(s_ref, regp_ref, loss_ref, reg_ref):
    s = s_ref[...]
    loss_ref[0, 0] = jnp.sum(jax.nn.softplus(s)) * (1.0 / B)
    reg_ref[0, 0] = jnp.sum(regp_ref[...]) * (0.5 / B)


def kernel(users, pos, neg, user_table, item_table):
    s, regp = _sc_call(users, pos, neg, user_table, item_table)
    loss, reg = pl.pallas_call(
        _tc_body,
        out_shape=(jax.ShapeDtypeStruct((1, 1), jnp.float32),
                   jax.ShapeDtypeStruct((1, 1), jnp.float32)),
        out_specs=(pl.BlockSpec(memory_space=pltpu.SMEM),
                   pl.BlockSpec(memory_space=pltpu.SMEM)),
    )(s.reshape(B // 128, 128), regp)
    return (loss[0, 0], reg[0, 0])


# speedup vs baseline: 2.5845x; 1.7053x over previous
import functools

import jax
import jax.numpy as jnp
from jax import lax
from jax.experimental import pallas as pl
from jax.experimental.pallas import tpu as pltpu
from jax.experimental.pallas import tpu_sc as plsc

B = 16384
D = 64
V = 1000000
L = 16
NC, NS = 2, 16
W = 512
VCUT = (V // W) * W
NWIN0 = VCUT // W
SENT = V
BKT = 4096
SBK = 768
HCAP = 1024


def _stream_core(tab_h, streams, s_idx, chunk, bidx, bk, sbidx, sbk, win,
                 whr, whk, stage, sk, semwin, semw):
    lanes = lax.iota(jnp.int32, L)
    sentv = jnp.full((L,), SENT, jnp.int32)

    def pre_b(v, _):
        bidx[pl.ds(v * L, L)] = sentv
        return 0
    lax.fori_loop(0, BKT // L, pre_b, 0)

    for sb in range(8):
        def pre_sb(v, _, sb=sb):
            sbidx[sb, pl.ds(v * L, L)] = sentv
            return 0
        lax.fori_loop(0, SBK // L, pre_sb, 0)

    def pre_w(v, _):
        z = jnp.zeros((L,), jnp.int32)
        whr[pl.ds(v * L, L)] = z
        whk[pl.ds(v * L, L)] = z
        return 0
    lax.fori_loop(0, HCAP // L, pre_w, 0)

    off = jnp.int32(0)
    for tag, idx_h, _out in streams:
        for ci in range(B // 2048):
            pltpu.sync_copy(idx_h.at[pl.ds(ci * 2048, 2048)], chunk)

            def p1(v, o, ci=ci, tag=tag):
                idx = chunk[pl.ds(v * L, L)]
                m = (((idx >> 9) & 15) == s_idx) & (idx < VCUT)
                dst = o + plsc.cumsum(m.astype(jnp.int32)) - 1
                plsc.store_scatter(bidx, [dst], idx, mask=m)
                kv = (ci * 2048 + v * L + tag) + lanes
                plsc.store_scatter(bk, [dst], kv, mask=m)
                return o + plsc.all_reduce_population_count(m)[0]

            off = lax.fori_loop(0, 2048 // L, p1, off)

    for sb in range(8):
        def p2(v, so, sb=sb):
            ivec = bidx[pl.ds(v * L, L)]
            kvec = bk[pl.ds(v * L, L)]
            m = (((ivec >> 13) & 7) == sb) & (ivec < VCUT)
            dst = so + plsc.cumsum(m.astype(jnp.int32)) - 1
            sbs = jnp.full((L,), sb, jnp.int32)
            plsc.store_scatter(sbidx, [sbs, dst], ivec, mask=m)
            plsc.store_scatter(sbk, [sbs, dst], kvec, mask=m)
            return so + plsc.all_reduce_population_count(m)[0]

        lax.fori_loop(0, BKT // L, p2, jnp.int32(0))

    nwin = jnp.where(s_idx == 0, (NWIN0 + 15) // 16, NWIN0 // 16)
    base0 = pl.multiple_of(s_idx * W, W)
    pltpu.async_copy(tab_h.at[pl.ds(0, D), pl.ds(base0, W)], win.at[0], semwin)

    outA_h, outB_h = streams[0][2], streams[-1][2]

    def window(l, twc):
        b = l & 1
        pltpu.make_async_copy(tab_h.at[pl.ds(0, D), pl.ds(0, W)],
                              win.at[b], semwin).wait()

        @pl.when(l + 1 < nwin)
        def _():
            g2 = (l + 1) * 16 + s_idx
            nb = pl.multiple_of(g2 * W, W)
            pltpu.async_copy(tab_h.at[pl.ds(0, D), pl.ds(nb, W)],
                             win.at[(l + 1) & 1], semwin)

        g = l * 16 + s_idx
        sb = l & 7

        def scan(v, wc):
            ivec = sbidx[sb, pl.ds(v * L, L)]
            kvec = sbk[sb, pl.ds(v * L, L)]
            m = (ivec >> 9) == g
            dst = wc + plsc.cumsum(m.astype(jnp.int32)) - 1
            plsc.store_scatter(whr, [dst], ivec & (W - 1), mask=m)
            plsc.store_scatter(whk, [dst], kvec, mask=m)
            return wc + plsc.all_reduce_population_count(m)[0]

        whc = lax.fori_loop(0, SBK // L, scan, jnp.int32(0))

        bs = jnp.full((L,), b, jnp.int32)

        def group(h, _):
            rel = whr[pl.ds(h * L, L)]
            kv = whk[pl.ds(h * L, L)]
            for d in range(D):
                dsp = jnp.full((L,), d, jnp.int32)
                vals = plsc.load_gather(win, [bs, dsp, rel])
                plsc.store_scatter(stage, [lanes, dsp], vals)
            for lane in range(L):
                sk[lane] = kv[lane]
                valid = h * L + lane < whc
                kk = sk[lane] & (B - 1)
                tag = sk[lane] >> 14

                @pl.when(valid & (tag == 0))
                def _():
                    pltpu.async_copy(stage.at[lane], outA_h.at[kk], semw)

                @pl.when(valid & (tag == 1))
                def _():
                    pltpu.async_copy(stage.at[lane], outB_h.at[kk], semw)
            cnt = jnp.minimum(jnp.int32(L), whc - h * L)

            def dr(i, _):
                pltpu.make_async_copy(outA_h.at[0], stage.at[0], semw).wait()
                return 0
            lax.fori_loop(0, cnt, dr, 0)
            return 0

        lax.fori_loop(0, (whc + L - 1) >> 4, group, 0)
        return twc + whc

    lax.fori_loop(0, nwin, window, jnp.int32(0))


def _sc_body(users_h, pos_h, neg_h, utT_h, itT_h, out_u, out_p, out_n,
             chunk, bidx, bk, sbidx, sbk, win, whr, whk, stage, sk,
             semwin, semw):
    c = lax.axis_index("c")
    s_idx = lax.axis_index("s")

    @pl.when(c == 0)
    def _():
        _stream_core(utT_h, [(0, users_h, out_u)], s_idx, chunk, bidx, bk,
                     sbidx, sbk, win, whr, whk, stage, sk, semwin, semw)

    @pl.when(c == 1)
    def _():
        _stream_core(itT_h, [(0, pos_h, out_p), (1 << 14, neg_h, out_n)],
                     s_idx, chunk, bidx, bk, sbidx, sbk, win, whr, whk,
                     stage, sk, semwin, semw)


_sc_call = functools.partial(
    pl.kernel,
    out_type=(jax.ShapeDtypeStruct((B, D), jnp.float32),
              jax.ShapeDtypeStruct((B, D), jnp.float32),
              jax.ShapeDtypeStruct((B, D), jnp.float32)),
    mesh=plsc.VectorSubcoreMesh(core_axis_name="c", subcore_axis_name="s",
                                num_cores=NC, num_subcores=NS),
    compiler_params=pltpu.CompilerParams(needs_layout_passes=False),
    scratch_types=(
        pltpu.VMEM((2048,), jnp.int32),
        pltpu.VMEM((BKT,), jnp.int32),
        pltpu.VMEM((BKT,), jnp.int32),
        pltpu.VMEM((8, SBK), jnp.int32),
        pltpu.VMEM((8, SBK), jnp.int32),
        pltpu.VMEM((2, D, W), jnp.float32),
        pltpu.VMEM((HCAP,), jnp.int32),
        pltpu.VMEM((HCAP,), jnp.int32),
        pltpu.VMEM((L, D), jnp.float32),
        pltpu.SMEM((L,), jnp.int32),
        pltpu.SemaphoreType.DMA,
        pltpu.SemaphoreType.DMA,
    ),
)(_sc_body)

GB = 16
RB = B // GB


def _tc_body(u_ref, p_ref, n_ref, ui_ref, pi_ref, ni_ref, tu_ref, ti_ref,
             loss_ref, reg_ref):
    step = pl.program_id(0)

    def fix(x, idx, tail):
        t = idx - VCUT
        cols = lax.broadcasted_iota(jnp.int32, (RB, D), 1)
        oh = (cols == t).astype(jnp.float32)
        fixed = jnp.dot(oh, tail, preferred_element_type=jnp.float32)
        return jnp.where(t >= 0, fixed, x)

    u = fix(u_ref[...], ui_ref[...], tu_ref[...])
    p = fix(p_ref[...], pi_ref[...], ti_ref[...])
    n = fix(n_ref[...], ni_ref[...], ti_ref[...])

    s = jnp.sum(u * (n - p), axis=1)
    part_loss = jnp.sum(jax.nn.softplus(s))
    part_reg = jnp.sum(u * u) + jnp.sum(p * p) + jnp.sum(n * n)

    @pl.when(step == 0)
    def _():
        loss_ref[0, 0] = 0.0
        reg_ref[0, 0] = 0.0

    loss_ref[0, 0] += part_loss * (1.0 / B)
    reg_ref[0, 0] += part_reg * (0.5 / B)


def kernel(users, pos, neg, user_table, item_table):
    utT = user_table.T
    itT = item_table.T
    tail_u = user_table[VCUT:, :]
    tail_i = item_table[VCUT:, :]
    out_u, out_p, out_n = _sc_call(users, pos, neg, utT, itT)

    row = lambda i: (i, 0)
    zero = lambda i: (0, 0)
    loss, reg = pl.pallas_call(
        _tc_body,
        grid=(GB,),
        in_specs=[
            pl.BlockSpec((RB, D), row),
            pl.BlockSpec((RB, D), row),
            pl.BlockSpec((RB, D), row),
            pl.BlockSpec((RB, 1), row),
            pl.BlockSpec((RB, 1), row),
            pl.BlockSpec((RB, 1), row),
            pl.BlockSpec((D, D), zero),
            pl.BlockSpec((D, D), zero),
        ],
        out_shape=(jax.ShapeDtypeStruct((1, 1), jnp.float32),
                   jax.ShapeDtypeStruct((1, 1), jnp.float32)),
        out_specs=(pl.BlockSpec(memory_space=pltpu.SMEM),
                   pl.BlockSpec(memory_space=pltpu.SMEM)),
    )(out_u, out_p, out_n,
      users.reshape(B, 1), pos.reshape(B, 1), neg.reshape(B, 1),
      tail_u, tail_i)
    return (loss[0, 0], reg[0, 0])
